# baseline (speedup 1.0000x reference)
import functools

import jax
import jax.numpy as jnp
from jax import lax
from jax.experimental import pallas as pl
from jax.experimental.pallas import tpu as pltpu
from jax.experimental.pallas import tpu_sc as plsc

INPUT_DIM = 49152
EMBED_DIM = 128
N_IDX = 819200

NC = 2
NS = 16
NW = NC * NS
LANES = 16

N_PER = N_IDX // NW
N_VECS = N_PER // LANES
ZERO_VECS = INPUT_DIM // LANES
UNROLL = 16


def _hist_body(idx_hbm, out_hbm, idx_v, counts_v, sem):
  wid = lax.axis_index("s") * NC + lax.axis_index("c")

  cp = pltpu.make_async_copy(
      idx_hbm.at[pl.ds(wid * N_PER, N_PER)], idx_v, sem)
  cp.start()

  zeros = jnp.zeros((LANES,), jnp.float32)
  def zbody(i, carry):
    for u in range(UNROLL):
      counts_v[pl.ds((i * UNROLL + u) * LANES, LANES)] = zeros
    return carry
  lax.fori_loop(0, ZERO_VECS // UNROLL, zbody, 0)

  cp.wait()

  ones = jnp.ones((LANES,), jnp.float32)
  def body(i, carry):
    base = i * (UNROLL * LANES)
    for u in range(UNROLL):
      iv = idx_v[pl.ds(base + u * LANES, LANES)]
      plsc.addupdate_scatter(counts_v, [iv], ones)
    return carry
  lax.fori_loop(0, N_VECS // UNROLL, body, 0)

  pltpu.sync_copy(counts_v, out_hbm.at[wid])


@functools.cache
def _hist():
  return functools.partial(
      pl.kernel,
      out_type=jax.ShapeDtypeStruct((NW, INPUT_DIM), jnp.float32),
      mesh=plsc.VectorSubcoreMesh(core_axis_name="c", subcore_axis_name="s",
                                  num_cores=NC, num_subcores=NS),
      compiler_params=pltpu.CompilerParams(needs_layout_passes=False),
      scratch_types=[
          pltpu.VMEM((N_PER,), jnp.int32),
          pltpu.VMEM((INPUT_DIM,), jnp.float32),
          pltpu.SemaphoreType.DMA,
      ],
  )(_hist_body)


K_BLOCKS = 4
ROW_BLK = INPUT_DIM // K_BLOCKS


def _reduce_body(p_ref, out_ref):
  out_ref[...] = jnp.sum(p_ref[...], axis=0, keepdims=True)


def _mlp_body(c_ref, t_ref, w1_ref, b1_ref, w2_ref, b2_ref, w3_ref, b3_ref,
              out_ref, acc_ref):
  k = pl.program_id(0)

  @pl.when(k == 0)
  def _():
    acc_ref[...] = jnp.zeros_like(acc_ref)

  acc_ref[...] += jnp.sum(c_ref[...] * t_ref[...], axis=0, keepdims=True)

  @pl.when(k == K_BLOCKS - 1)
  def _():
    s = acc_ref[...]
    h1 = jnp.maximum(
        jnp.dot(s, w1_ref[...], preferred_element_type=jnp.float32,
                precision=lax.Precision.HIGHEST) + b1_ref[...], 0.0)
    h2 = jnp.maximum(
        jnp.dot(h1, w2_ref[...], preferred_element_type=jnp.float32,
                precision=lax.Precision.HIGHEST) + b2_ref[...], 0.0)
    o = jnp.sum(h2 * w3_ref[...], axis=1, keepdims=True) + b3_ref[...]
    out_ref[...] = jnp.tanh(o)


def kernel(indices, table, W1, b1, W2, b2, W3, b3):
  partials = _hist()(indices)

  counts = pl.pallas_call(
      _reduce_body,
      grid=(K_BLOCKS,),
      in_specs=[pl.BlockSpec((NW, ROW_BLK), lambda k: (0, k))],
      out_specs=pl.BlockSpec((1, ROW_BLK), lambda k: (0, k)),
      out_shape=jax.ShapeDtypeStruct((1, INPUT_DIM), jnp.float32),
  )(partials)

  out = pl.pallas_call(
      _mlp_body,
      grid=(K_BLOCKS,),
      in_specs=[
          pl.BlockSpec((ROW_BLK, 1), lambda k: (k, 0)),
          pl.BlockSpec((ROW_BLK, EMBED_DIM), lambda k: (k, 0)),
          pl.BlockSpec((EMBED_DIM, 32), lambda k: (0, 0)),
          pl.BlockSpec((1, 32), lambda k: (0, 0)),
          pl.BlockSpec((32, 32), lambda k: (0, 0)),
          pl.BlockSpec((1, 32), lambda k: (0, 0)),
          pl.BlockSpec((1, 32), lambda k: (0, 0)),
          pl.BlockSpec((1, 1), lambda k: (0, 0)),
      ],
      out_specs=pl.BlockSpec((1, 1), lambda k: (0, 0)),
      out_shape=jax.ShapeDtypeStruct((1, 1), jnp.float32),
      scratch_shapes=[pltpu.VMEM((1, EMBED_DIM), jnp.float32)],
  )(counts.reshape(INPUT_DIM, 1), table, W1, b1.reshape(1, 32), W2,
    b2.reshape(1, 32), W3.reshape(1, 32), b3.reshape(1, 1))

  return out.reshape(())


# speedup vs baseline: 1.3683x; 1.3683x over previous
import functools

import jax
import jax.numpy as jnp
from jax import lax
from jax.experimental import pallas as pl
from jax.experimental.pallas import tpu as pltpu
from jax.experimental.pallas import tpu_sc as plsc

INPUT_DIM = 49152
EMBED_DIM = 128
N_IDX = 819200

NC = 2
NS = 16
NW = NC * NS
LANES = 16

N_PER = N_IDX // NW
N_VECS = N_PER // LANES
ZERO_VECS = INPUT_DIM // LANES
UNROLL = 16


def _hist_body(idx_hbm, out_hbm, idx_v, counts_v, sem):
  wid = lax.axis_index("s") * NC + lax.axis_index("c")

  cp = pltpu.make_async_copy(
      idx_hbm.at[pl.ds(wid * N_PER, N_PER)], idx_v, sem)
  cp.start()

  zeros = jnp.zeros((LANES,), jnp.float32)
  def zbody(i, carry):
    for u in range(UNROLL):
      counts_v[pl.ds((i * UNROLL + u) * LANES, LANES)] = zeros
    return carry
  lax.fori_loop(0, ZERO_VECS // UNROLL, zbody, 0)

  cp.wait()

  ones = jnp.ones((LANES,), jnp.float32)
  def body(i, carry):
    base = i * (UNROLL * LANES)
    for u in range(UNROLL):
      iv = idx_v[pl.ds(base + u * LANES, LANES)]
      plsc.addupdate_scatter(counts_v, [iv], ones)
    return carry
  lax.fori_loop(0, N_VECS // UNROLL, body, 0)

  pltpu.sync_copy(counts_v, out_hbm.at[wid])


@functools.cache
def _hist():
  return functools.partial(
      pl.kernel,
      out_type=jax.ShapeDtypeStruct((NW, INPUT_DIM), jnp.float32),
      mesh=plsc.VectorSubcoreMesh(core_axis_name="c", subcore_axis_name="s",
                                  num_cores=NC, num_subcores=NS),
      compiler_params=pltpu.CompilerParams(needs_layout_passes=False),
      scratch_types=[
          pltpu.VMEM((N_PER,), jnp.int32),
          pltpu.VMEM((INPUT_DIM,), jnp.float32),
          pltpu.SemaphoreType.DMA,
      ],
  )(_hist_body)


K_BLOCKS = 8
ROW_BLK = INPUT_DIM // K_BLOCKS


def _mlp_body(p_ref, t_ref, w1_ref, b1_ref, w2_ref, b2_ref, w3_ref, b3_ref,
              out_ref, acc_ref):
  k = pl.program_id(0)

  @pl.when(k == 0)
  def _():
    acc_ref[...] = jnp.zeros_like(acc_ref)

  c = jnp.sum(p_ref[...], axis=0, keepdims=True)
  acc_ref[...] += jnp.dot(c, t_ref[...], preferred_element_type=jnp.float32,
                          precision=lax.Precision.HIGHEST)

  @pl.when(k == K_BLOCKS - 1)
  def _():
    s = acc_ref[...]
    h1 = jnp.maximum(
        jnp.dot(s, w1_ref[...], preferred_element_type=jnp.float32,
                precision=lax.Precision.HIGHEST) + b1_ref[...], 0.0)
    h2 = jnp.maximum(
        jnp.dot(h1, w2_ref[...], preferred_element_type=jnp.float32,
                precision=lax.Precision.HIGHEST) + b2_ref[...], 0.0)
    o = jnp.sum(h2 * w3_ref[...], axis=1, keepdims=True) + b3_ref[...]
    out_ref[...] = jnp.tanh(o)


def kernel(indices, table, W1, b1, W2, b2, W3, b3):
  partials = _hist()(indices)

  out = pl.pallas_call(
      _mlp_body,
      grid=(K_BLOCKS,),
      in_specs=[
          pl.BlockSpec((NW, ROW_BLK), lambda k: (0, k)),
          pl.BlockSpec((ROW_BLK, EMBED_DIM), lambda k: (k, 0)),
          pl.BlockSpec((EMBED_DIM, 32), lambda k: (0, 0)),
          pl.BlockSpec((1, 32), lambda k: (0, 0)),
          pl.BlockSpec((32, 32), lambda k: (0, 0)),
          pl.BlockSpec((1, 32), lambda k: (0, 0)),
          pl.BlockSpec((1, 32), lambda k: (0, 0)),
          pl.BlockSpec((1, 1), lambda k: (0, 0)),
      ],
      out_specs=pl.BlockSpec((1, 1), lambda k: (0, 0)),
      out_shape=jax.ShapeDtypeStruct((1, 1), jnp.float32),
      scratch_shapes=[pltpu.VMEM((1, EMBED_DIM), jnp.float32)],
  )(partials, table, W1, b1.reshape(1, 32), W2, b2.reshape(1, 32),
    W3.reshape(1, 32), b3.reshape(1, 1))

  return out.reshape(())


# speedup vs baseline: 1.3976x; 1.0215x over previous
import functools

import jax
import jax.numpy as jnp
from jax import lax
from jax.experimental import pallas as pl
from jax.experimental.pallas import tpu as pltpu
from jax.experimental.pallas import tpu_sc as plsc

INPUT_DIM = 49152
EMBED_DIM = 128
N_IDX = 819200

NC = 2
NS = 16
NW = NC * NS
LANES = 16

N_PER = N_IDX // NW
N_VECS = N_PER // LANES
ZERO_VECS = INPUT_DIM // LANES
UNROLL = 16


def _hist_body(idx_hbm, out_hbm, idx_v, counts_v, sem):
  wid = lax.axis_index("s") * NC + lax.axis_index("c")

  cp = pltpu.make_async_copy(
      idx_hbm.at[pl.ds(wid * N_PER, N_PER)], idx_v, sem)
  cp.start()

  with jax.named_scope("hist_zero"):
    zeros = jnp.zeros((LANES,), jnp.float32)
    def zbody(i, carry):
      for u in range(UNROLL):
        counts_v[pl.ds((i * UNROLL + u) * LANES, LANES)] = zeros
      return carry
    lax.fori_loop(0, ZERO_VECS // UNROLL, zbody, 0)

  with jax.named_scope("idx_wait"):
    cp.wait()

  with jax.named_scope("hist_scatter"):
    ones = jnp.ones((LANES,), jnp.float32)
    def body(i, carry):
      base = i * (UNROLL * LANES)
      for u in range(UNROLL):
        iv = idx_v[pl.ds(base + u * LANES, LANES)]
        plsc.addupdate_scatter(counts_v, [iv], ones)
      return carry
    lax.fori_loop(0, N_VECS // UNROLL, body, 0)

  with jax.named_scope("hist_writeback"):
    pltpu.sync_copy(counts_v, out_hbm.at[wid])


@functools.cache
def _hist():
  return functools.partial(
      pl.kernel,
      out_type=jax.ShapeDtypeStruct((NW, INPUT_DIM), jnp.float32),
      mesh=plsc.VectorSubcoreMesh(core_axis_name="c", subcore_axis_name="s",
                                  num_cores=NC, num_subcores=NS),
      compiler_params=pltpu.CompilerParams(needs_layout_passes=False),
      scratch_types=[
          pltpu.VMEM((N_PER,), jnp.int32),
          pltpu.VMEM((INPUT_DIM,), jnp.float32),
          pltpu.SemaphoreType.DMA,
      ],
  )(_hist_body)


K_BLOCKS = 4
ROW_BLK = INPUT_DIM // K_BLOCKS


def _mlp_body(p_ref, t_ref, w1_ref, b1_ref, w2_ref, b2_ref, w3_ref, b3_ref,
              out_ref, acc_ref):
  k = pl.program_id(0)

  @pl.when(k == 0)
  def _():
    acc_ref[...] = jnp.zeros_like(acc_ref)

  c = jnp.sum(p_ref[...], axis=0, keepdims=True)
  acc_ref[...] += jnp.dot(c, t_ref[...], preferred_element_type=jnp.float32,
                          precision=lax.Precision.HIGHEST)

  @pl.when(k == K_BLOCKS - 1)
  def _():
    s = acc_ref[...]
    h1 = jnp.maximum(
        jnp.dot(s, w1_ref[...], preferred_element_type=jnp.float32,
                precision=lax.Precision.HIGHEST) + b1_ref[...], 0.0)
    h2 = jnp.maximum(
        jnp.dot(h1, w2_ref[...], preferred_element_type=jnp.float32,
                precision=lax.Precision.HIGHEST) + b2_ref[...], 0.0)
    o = jnp.sum(h2 * w3_ref[...], axis=1, keepdims=True) + b3_ref[...]
    out_ref[...] = jnp.tanh(o)


def kernel(indices, table, W1, b1, W2, b2, W3, b3):
  partials = _hist()(indices)

  out = pl.pallas_call(
      _mlp_body,
      grid=(K_BLOCKS,),
      in_specs=[
          pl.BlockSpec((NW, ROW_BLK), lambda k: (0, k)),
          pl.BlockSpec((ROW_BLK, EMBED_DIM), lambda k: (k, 0)),
          pl.BlockSpec((EMBED_DIM, 32), lambda k: (0, 0)),
          pl.BlockSpec((1, 32), lambda k: (0, 0)),
          pl.BlockSpec((32, 32), lambda k: (0, 0)),
          pl.BlockSpec((1, 32), lambda k: (0, 0)),
          pl.BlockSpec((1, 32), lambda k: (0, 0)),
          pl.BlockSpec((1, 1), lambda k: (0, 0)),
      ],
      out_specs=pl.BlockSpec((1, 1), lambda k: (0, 0)),
      out_shape=jax.ShapeDtypeStruct((1, 1), jnp.float32),
      scratch_shapes=[pltpu.VMEM((1, EMBED_DIM), jnp.float32)],
  )(partials, table, W1, b1.reshape(1, 32), W2, b2.reshape(1, 32),
    W3.reshape(1, 32), b3.reshape(1, 1))

  return out.reshape(())


# speedup vs baseline: 1.7502x; 1.2522x over previous
import functools

import jax
import jax.numpy as jnp
from jax import lax
from jax.experimental import pallas as pl
from jax.experimental.pallas import tpu as pltpu
from jax.experimental.pallas import tpu_sc as plsc

INPUT_DIM = 49152
EMBED_DIM = 128
N_IDX = 819200

NC = 2
NS = 16
NW = NC * NS
LANES = 16

N_PER = N_IDX // NW
N_VECS = N_PER // LANES
ZERO_VECS = INPUT_DIM // LANES
UNROLL = 16


def _hist_body(idx_hbm, out_hbm, idx_v, counts_v, sem):
  wid = lax.axis_index("s") * NC + lax.axis_index("c")

  cp = pltpu.make_async_copy(
      idx_hbm.at[pl.ds(wid * N_PER, N_PER)], idx_v, sem)
  cp.start()

  with jax.named_scope("hist_zero"):
    zeros = jnp.zeros((LANES,), jnp.float32)
    @functools.partial(plsc.parallel_loop, 0, ZERO_VECS, unroll=UNROLL)
    def _(i):
      counts_v[pl.ds(i * LANES, LANES)] = zeros

  with jax.named_scope("idx_wait"):
    cp.wait()

  with jax.named_scope("hist_scatter"):
    ones = jnp.ones((LANES,), jnp.float32)
    @functools.partial(plsc.parallel_loop, 0, N_VECS, unroll=UNROLL)
    def _(i):
      iv = idx_v[pl.ds(i * LANES, LANES)]
      plsc.addupdate_scatter(counts_v, [iv], ones)

  with jax.named_scope("hist_writeback"):
    pltpu.sync_copy(counts_v, out_hbm.at[wid])


@functools.cache
def _hist():
  return functools.partial(
      pl.kernel,
      out_type=jax.ShapeDtypeStruct((NW, INPUT_DIM), jnp.float32),
      mesh=plsc.VectorSubcoreMesh(core_axis_name="c", subcore_axis_name="s",
                                  num_cores=NC, num_subcores=NS),
      compiler_params=pltpu.CompilerParams(needs_layout_passes=False),
      scratch_types=[
          pltpu.VMEM((N_PER,), jnp.int32),
          pltpu.VMEM((INPUT_DIM,), jnp.float32),
          pltpu.SemaphoreType.DMA,
      ],
  )(_hist_body)


K_BLOCKS = 4
ROW_BLK = INPUT_DIM // K_BLOCKS


def _mlp_body(p_ref, t_ref, w1_ref, b1_ref, w2_ref, b2_ref, w3_ref, b3_ref,
              out_ref, acc_ref):
  k = pl.program_id(0)

  @pl.when(k == 0)
  def _():
    acc_ref[...] = jnp.zeros_like(acc_ref)

  c = jnp.sum(p_ref[...], axis=0, keepdims=True)
  acc_ref[...] += jnp.dot(c, t_ref[...], preferred_element_type=jnp.float32,
                          precision=lax.Precision.HIGHEST)

  @pl.when(k == K_BLOCKS - 1)
  def _():
    s = acc_ref[...]
    h1 = jnp.maximum(
        jnp.dot(s, w1_ref[...], preferred_element_type=jnp.float32,
                precision=lax.Precision.HIGHEST) + b1_ref[...], 0.0)
    h2 = jnp.maximum(
        jnp.dot(h1, w2_ref[...], preferred_element_type=jnp.float32,
                precision=lax.Precision.HIGHEST) + b2_ref[...], 0.0)
    o = jnp.sum(h2 * w3_ref[...], axis=1, keepdims=True) + b3_ref[...]
    out_ref[...] = jnp.tanh(o)


def kernel(indices, table, W1, b1, W2, b2, W3, b3):
  partials = _hist()(indices)

  out = pl.pallas_call(
      _mlp_body,
      grid=(K_BLOCKS,),
      in_specs=[
          pl.BlockSpec((NW, ROW_BLK), lambda k: (0, k)),
          pl.BlockSpec((ROW_BLK, EMBED_DIM), lambda k: (k, 0)),
          pl.BlockSpec((EMBED_DIM, 32), lambda k: (0, 0)),
          pl.BlockSpec((1, 32), lambda k: (0, 0)),
          pl.BlockSpec((32, 32), lambda k: (0, 0)),
          pl.BlockSpec((1, 32), lambda k: (0, 0)),
          pl.BlockSpec((1, 32), lambda k: (0, 0)),
          pl.BlockSpec((1, 1), lambda k: (0, 0)),
      ],
      out_specs=pl.BlockSpec((1, 1), lambda k: (0, 0)),
      out_shape=jax.ShapeDtypeStruct((1, 1), jnp.float32),
      scratch_shapes=[pltpu.VMEM((1, EMBED_DIM), jnp.float32)],
  )(partials, table, W1, b1.reshape(1, 32), W2, b2.reshape(1, 32),
    W3.reshape(1, 32), b3.reshape(1, 1))

  return out.reshape(())
